# baseline (speedup 1.0000x reference)
import functools

import jax
import jax.numpy as jnp
from jax import lax
from jax.experimental import pallas as pl
from jax.experimental.pallas import tpu as pltpu
from jax.experimental.pallas import tpu_sc as plsc

_NC = 2
_NS = 16
_NW = _NC * _NS
_L = 16
_CHUNK = 512
_GROUP = 128
_FLUSH = 1024
_PCAP = 2128
_PEND_MAX = 1536


def _pad_edges(E):
    step = _NW * _CHUNK
    return ((E + step - 1) // step) * step


def _make_part_kernel(EP, NA):
    RPW = NA // _NW
    TRASH = RPW
    SENT = TRASH << 14
    NCH = EP // _CHUNK
    PWCH = NCH // _NW
    HR = NA // _L
    CAPR = EP + _PEND_MAX
    assert EP % (_CHUNK * _NW) == 0 and NA % _NW == 0 and RPW % 8 == 0

    mesh = plsc.VectorSubcoreMesh(core_axis_name="c", subcore_axis_name="s")

    @functools.partial(
        pl.kernel,
        out_type=(
            jax.ShapeDtypeStruct((_NW, 1, CAPR), jnp.int32),
            jax.ShapeDtypeStruct((_NW, 1, _L), jnp.int32),
            jax.ShapeDtypeStruct((_NW, HR + _L, _L), jnp.float32),
        ),
        mesh=mesh,
        compiler_params=pltpu.CompilerParams(needs_layout_passes=False),
        scratch_types=[
            pltpu.VMEM((_CHUNK,), jnp.int32),
            pltpu.VMEM((_CHUNK,), jnp.int32),
            pltpu.VMEM((_PCAP,), jnp.int32),
            pltpu.VMEM((HR + _L, _L), jnp.float32),
            pltpu.VMEM((_L,), jnp.int32),
        ],
    )
    def part_kernel(row_hbm, col_hbm, plist_hbm, cnt_hbm, deg_hbm,
                    rowb, colb, pend, hist, cntb):
        c = lax.axis_index("c")
        s = lax.axis_index("s")
        w = s * _NC + c
        lo = w * RPW
        zero16 = jnp.zeros((_L,), jnp.float32)
        one = jnp.float32(1.0)
        iota = lax.iota(jnp.int32, _L)

        def zbody(r, carry):
            hist[r, :] = zero16
            return carry

        lax.fori_loop(0, HR + _L, zbody, 0)

        def body(i, carry):
            ptr, hptr = carry
            base = pl.multiple_of(i * _CHUNK, 8)
            pltpu.sync_copy(row_hbm.at[pl.ds(base, _CHUNK)], rowb)
            pltpu.sync_copy(col_hbm.at[pl.ds(base, _CHUNK)], colb)

            def sbody(j, p):
                cv = colb[pl.ds(j * _L, _L)]
                rv = rowb[pl.ds(j * _L, _L)]
                lv = cv - lo
                m = (lv >= 0) & (lv < RPW)
                key = jnp.where(m, 0, 1)
                pv = jnp.where(m, rv | (lv << 14), SENT)
                _, sv = plsc.sort_key_val(key, pv)
                pend[pl.ds(p, _L)] = sv
                return p + plsc.all_reduce_population_count(m)[0]

            ptr = lax.fori_loop(0, _CHUNK // _L, sbody, ptr, unroll=4)

            @pl.when((i >= w * PWCH) & (i < (w + 1) * PWCH))
            def _():
                def dbody(j, carry2):
                    cv = colb[pl.ds(j * _L, _L)]
                    for e in range(_L):
                        v = cv[e]
                        onehot = jnp.where(iota == (v & (_L - 1)), one, 0.0)
                        plsc.addupdate(hist.at[v >> 4], onehot)
                    return carry2

                lax.fori_loop(0, _CHUNK // _L, dbody, 0, unroll=2)

            def flush(args):
                p, h = args
                hb = pl.multiple_of(h, 128)
                pltpu.sync_copy(pend.at[pl.ds(0, _FLUSH)],
                                plist_hbm.at[w, 0, pl.ds(hb, _FLUSH)])
                rem = p - _FLUSH
                nrem = (rem + _L - 1) // _L

                def mbody(t, carry3):
                    src = pl.multiple_of(_FLUSH + t * _L, 8)
                    dst = pl.multiple_of(t * _L, 8)
                    pend[pl.ds(dst, _L)] = pend[pl.ds(src, _L)]
                    return carry3

                lax.fori_loop(0, nrem, mbody, 0)
                return rem, h + _FLUSH

            ptr, hptr = lax.cond(ptr >= _FLUSH, flush,
                                 lambda a: a, (ptr, hptr))
            return ptr, hptr

        ptr, hptr = lax.fori_loop(0, NCH, body,
                                  (jnp.int32(0), jnp.int32(0)))

        ngroups = (ptr + _GROUP - 1) // _GROUP
        pend_end = ngroups * _GROUP
        fl = pl.multiple_of((ptr // _L) * _L, 8)
        lane = ptr - fl
        keep = iota < lane
        pend[pl.ds(fl, _L)] = jnp.where(keep, pend[pl.ds(fl, _L)],
                                        jnp.int32(SENT))

        def fbody(t, carry):
            dst = pl.multiple_of(fl + _L + t * _L, 8)
            pend[pl.ds(dst, _L)] = jnp.full((_L,), SENT, jnp.int32)
            return carry

        lax.fori_loop(0, (_PEND_MAX - fl) // _L, fbody, 0)
        hb = pl.multiple_of(hptr, 128)
        pltpu.sync_copy(pend.at[pl.ds(0, _PEND_MAX)],
                        plist_hbm.at[w, 0, pl.ds(hb, _PEND_MAX)])
        total = hptr + pend_end
        cntb[...] = total + jnp.zeros((_L,), jnp.int32)
        pltpu.sync_copy(cntb, cnt_hbm.at[w, 0])
        pltpu.sync_copy(hist, deg_hbm.at[w])

    return part_kernel


_SG = 64


def _make_scatter_kernel(EP, NA, D):
    RPW = NA // _NW
    CAPR = EP + _PEND_MAX
    assert NA % _NW == 0 and RPW % 8 == 0 and _GROUP % _SG == 0

    mesh = plsc.VectorSubcoreMesh(core_axis_name="c", subcore_axis_name="s")

    @functools.partial(
        pl.kernel,
        out_type=jax.ShapeDtypeStruct((NA, D), jnp.float32),
        mesh=mesh,
        compiler_params=pltpu.CompilerParams(needs_layout_passes=False),
        scratch_types=[
            pltpu.VMEM((NA // _NW + 8, D), jnp.float32),
            pltpu.VMEM((_SG,), jnp.int32),
            pltpu.VMEM((_SG,), jnp.int32),
            pltpu.VMEM((_SG,), jnp.int32),
            pltpu.VMEM((_SG,), jnp.int32),
            pltpu.VMEM((_SG, D), jnp.float32),
            pltpu.VMEM((_SG, D), jnp.float32),
            pltpu.VMEM((_L,), jnp.int32),
            pltpu.SemaphoreType.DMA,
            pltpu.SemaphoreType.DMA,
        ],
    )
    def scat_kernel(hp_hbm, plist_hbm, cnt_hbm, out_hbm,
                    acc, pgA, pgB, idxA, idxB, gbA, gbB, cbuf, semA, semB):
        c = lax.axis_index("c")
        s = lax.axis_index("s")
        w = s * _NC + c
        lo = w * RPW
        zero16 = jnp.zeros((_L,), jnp.float32)

        def zbody(r, carry):
            for t in range(D // _L):
                acc[r, pl.ds(t * _L, _L)] = zero16
            return carry

        lax.fori_loop(0, RPW + 8, zbody, 0)

        pltpu.sync_copy(cnt_hbm.at[w, 0], cbuf)
        total = cbuf[...][0]
        ngroups = total // _SG

        pg = (pgA, pgB)
        idx = (idxA, idxB)
        gb = (gbA, gbB)
        sem = (semA, semB)

        def fire(g, k):
            o = pl.multiple_of(g * _SG, 64)
            pltpu.sync_copy(plist_hbm.at[w, 0, pl.ds(o, _SG)], pg[k])
            for t in range(_SG // _L):
                pv16 = pg[k][pl.ds(t * _L, _L)]
                idx[k][pl.ds(t * _L, _L)] = pv16 & 16383
            pltpu.async_copy(hp_hbm.at[idx[k]], gb[k], sem[k])

        def wait(k):
            pltpu.make_async_copy(hp_hbm.at[idx[k]], gb[k], sem[k]).wait()

        def accum(k):
            def ebody(u, carry2):
                sb = u * _L
                lvv = pg[k][pl.ds(sb, _L)] >> 14
                for e in range(_L):
                    lv = lvv[e]
                    ge = sb + e
                    for t in range(D // _L):
                        plsc.addupdate(acc.at[lv, pl.ds(t * _L, _L)],
                                       gb[k][ge, pl.ds(t * _L, _L)])
                return carry2

            lax.fori_loop(0, _SG // _L, ebody, 0, unroll=2)

        @pl.when(ngroups > 0)
        def _():
            fire(0, 0)

            def pbody(p, carry):
                g0 = p * 2
                g1 = g0 + 1
                wait(0)

                @pl.when(g1 < ngroups)
                def _():
                    fire(g1, 1)

                accum(0)

                @pl.when(g1 < ngroups)
                def _():
                    wait(1)

                    @pl.when(g1 + 1 < ngroups)
                    def _():
                        fire(g1 + 1, 0)

                    accum(1)

                return carry

            lax.fori_loop(0, (ngroups + 1) // 2, pbody, 0)

        pltpu.sync_copy(acc.at[pl.ds(0, RPW)], out_hbm.at[pl.ds(lo, RPW)])

    return scat_kernel


def _tc1_body(x_ref, w_ref, deg_ref, h1p_ref, dinv_ref):
    dinv = lax.rsqrt(deg_ref[...] + 1.0)
    h = jnp.dot(x_ref[...], w_ref[...], preferred_element_type=jnp.float32)
    h1p_ref[...] = h * dinv
    dinv_ref[...] = dinv


def _tc2_body(agg_ref, h1p_ref, dinv_ref, b0_ref, w1_ref, h2p_ref):
    dinv = dinv_ref[...]
    t = (agg_ref[...] + h1p_ref[...]) * dinv + b0_ref[...]
    z = jnp.maximum(t, 0.0)
    h2p_ref[...] = jnp.dot(z, w1_ref[...],
                           preferred_element_type=jnp.float32) * dinv


def _tc3_body(agg_ref, h2p_ref, dinv_ref, b1_ref, out_ref):
    out_ref[...] = (agg_ref[...] + h2p_ref[...]) * dinv_ref[...] + b1_ref[...]


def kernel(x, edge_index, W0, b0, W1, b1):
    N, Din = x.shape
    E = edge_index.shape[1]
    Dh = W0.shape[1]
    Dout = W1.shape[1]

    NA = ((N + _NW * 8 - 1) // (_NW * 8)) * (_NW * 8)
    EP = _pad_edges(E)
    pad = EP - E
    row = jnp.concatenate([edge_index[0], jnp.zeros((pad,), jnp.int32)])
    col = jnp.concatenate([edge_index[1], jnp.full((pad,), NA, jnp.int32)])

    part_kernel = _make_part_kernel(EP, NA)
    scat1 = _make_scatter_kernel(EP, NA, Dh)
    scat2 = _make_scatter_kernel(EP, NA, Dout)

    plist, cnts, dhists = part_kernel(row, col)
    degs = (jnp.sum(dhists, axis=0)[:NA // _L, :]
            .reshape(NA)[:N].reshape(N, 1))

    BM = 1000
    assert N % BM == 0
    grid = (N // BM,)

    h1p, dinv = pl.pallas_call(
        _tc1_body,
        grid=grid,
        in_specs=[
            pl.BlockSpec((BM, Din), lambda i: (i, 0)),
            pl.BlockSpec((Din, Dh), lambda i: (0, 0)),
            pl.BlockSpec((BM, 1), lambda i: (i, 0)),
        ],
        out_specs=[
            pl.BlockSpec((BM, Dh), lambda i: (i, 0)),
            pl.BlockSpec((BM, 1), lambda i: (i, 0)),
        ],
        out_shape=[
            jax.ShapeDtypeStruct((N, Dh), jnp.float32),
            jax.ShapeDtypeStruct((N, 1), jnp.float32),
        ],
    )(x, W0, degs)

    agg1 = scat1(h1p, plist, cnts)

    h2p = pl.pallas_call(
        _tc2_body,
        grid=grid,
        in_specs=[
            pl.BlockSpec((BM, Dh), lambda i: (i, 0)),
            pl.BlockSpec((BM, Dh), lambda i: (i, 0)),
            pl.BlockSpec((BM, 1), lambda i: (i, 0)),
            pl.BlockSpec((1, Dh), lambda i: (0, 0)),
            pl.BlockSpec((Dh, Dout), lambda i: (0, 0)),
        ],
        out_specs=pl.BlockSpec((BM, Dout), lambda i: (i, 0)),
        out_shape=jax.ShapeDtypeStruct((N, Dout), jnp.float32),
    )(agg1, h1p, dinv, b0.reshape(1, Dh), W1)

    agg2 = scat2(h2p, plist, cnts)

    out = pl.pallas_call(
        _tc3_body,
        grid=grid,
        in_specs=[
            pl.BlockSpec((BM, Dout), lambda i: (i, 0)),
            pl.BlockSpec((BM, Dout), lambda i: (i, 0)),
            pl.BlockSpec((BM, 1), lambda i: (i, 0)),
            pl.BlockSpec((1, Dout), lambda i: (0, 0)),
        ],
        out_specs=pl.BlockSpec((BM, Dout), lambda i: (i, 0)),
        out_shape=jax.ShapeDtypeStruct((N, Dout), jnp.float32),
    )(agg2, h2p, dinv, b1.reshape(1, Dout))

    return out


# speedup vs baseline: 1.1343x; 1.1343x over previous
import functools

import jax
import jax.numpy as jnp
from jax import lax
from jax.experimental import pallas as pl
from jax.experimental.pallas import tpu as pltpu
from jax.experimental.pallas import tpu_sc as plsc

_NC = 2
_NS = 16
_NW = _NC * _NS
_L = 16
_CHUNK = 512
_GROUP = 128
_FLUSH = 1024
_PCAP = 2128
_PEND_MAX = 1536


def _pad_edges(E):
    step = _NW * _CHUNK
    return ((E + step - 1) // step) * step


def _make_part_kernel(EP, NA):
    RPW = NA // _NW
    TRASH = RPW
    SENT = TRASH << 14
    NCH = EP // _CHUNK
    PWCH = NCH // _NW
    HR = NA // _L
    CAPR = EP + _PEND_MAX
    assert EP % (_CHUNK * _NW) == 0 and NA % _NW == 0 and RPW % 8 == 0

    mesh = plsc.VectorSubcoreMesh(core_axis_name="c", subcore_axis_name="s")

    @functools.partial(
        pl.kernel,
        out_type=(
            jax.ShapeDtypeStruct((_NW, 1, CAPR), jnp.int32),
            jax.ShapeDtypeStruct((_NW, 1, _L), jnp.int32),
            jax.ShapeDtypeStruct((_NW, HR + _L, _L), jnp.float32),
        ),
        mesh=mesh,
        compiler_params=pltpu.CompilerParams(needs_layout_passes=False),
        scratch_types=[
            pltpu.VMEM((_CHUNK,), jnp.int32),
            pltpu.VMEM((_CHUNK,), jnp.int32),
            pltpu.VMEM((_PCAP,), jnp.int32),
            pltpu.VMEM((HR + _L, _L), jnp.float32),
            pltpu.VMEM((_L,), jnp.int32),
        ],
    )
    def part_kernel(row_hbm, col_hbm, plist_hbm, cnt_hbm, deg_hbm,
                    rowb, colb, pend, hist, cntb):
        c = lax.axis_index("c")
        s = lax.axis_index("s")
        w = s * _NC + c
        lo = w * RPW
        zero16 = jnp.zeros((_L,), jnp.float32)
        one = jnp.float32(1.0)
        iota = lax.iota(jnp.int32, _L)

        def zbody(r, carry):
            hist[r, :] = zero16
            return carry

        lax.fori_loop(0, HR + _L, zbody, 0)

        def body(i, carry):
            ptr, hptr = carry
            base = pl.multiple_of(i * _CHUNK, 8)
            pltpu.sync_copy(row_hbm.at[pl.ds(base, _CHUNK)], rowb)
            pltpu.sync_copy(col_hbm.at[pl.ds(base, _CHUNK)], colb)

            def sbody(j, p):
                cv = colb[pl.ds(j * _L, _L)]
                lv = cv - lo
                m = (lv >= 0) & (lv < RPW)
                cnt = plsc.all_reduce_population_count(m)[0]

                @pl.when(cnt > 0)
                def _():
                    rv = rowb[pl.ds(j * _L, _L)]
                    key = jnp.where(m, 0, 1)
                    pv = jnp.where(m, rv | (lv << 14), SENT)
                    _, sv = plsc.sort_key_val(key, pv)
                    pend[pl.ds(p, _L)] = sv

                return p + cnt

            ptr = lax.fori_loop(0, _CHUNK // _L, sbody, ptr)

            @pl.when((i >= w * PWCH) & (i < (w + 1) * PWCH))
            def _():
                def dbody(j, carry2):
                    cv = colb[pl.ds(j * _L, _L)]
                    for e in range(_L):
                        v = cv[e]
                        onehot = jnp.where(iota == (v & (_L - 1)), one, 0.0)
                        plsc.addupdate(hist.at[v >> 4], onehot)
                    return carry2

                lax.fori_loop(0, _CHUNK // _L, dbody, 0)

            def flush(args):
                p, h = args
                hb = pl.multiple_of(h, 128)
                pltpu.sync_copy(pend.at[pl.ds(0, _FLUSH)],
                                plist_hbm.at[w, 0, pl.ds(hb, _FLUSH)])
                rem = p - _FLUSH
                nrem = (rem + _L - 1) // _L

                def mbody(t, carry3):
                    src = pl.multiple_of(_FLUSH + t * _L, 8)
                    dst = pl.multiple_of(t * _L, 8)
                    pend[pl.ds(dst, _L)] = pend[pl.ds(src, _L)]
                    return carry3

                lax.fori_loop(0, nrem, mbody, 0)
                return rem, h + _FLUSH

            ptr, hptr = lax.cond(ptr >= _FLUSH, flush,
                                 lambda a: a, (ptr, hptr))
            return ptr, hptr

        ptr, hptr = lax.fori_loop(0, NCH, body,
                                  (jnp.int32(0), jnp.int32(0)))

        ngroups = (ptr + _GROUP - 1) // _GROUP
        pend_end = ngroups * _GROUP
        fl = pl.multiple_of((ptr // _L) * _L, 8)
        lane = ptr - fl
        keep = iota < lane
        pend[pl.ds(fl, _L)] = jnp.where(keep, pend[pl.ds(fl, _L)],
                                        jnp.int32(SENT))

        def fbody(t, carry):
            dst = pl.multiple_of(fl + _L + t * _L, 8)
            pend[pl.ds(dst, _L)] = jnp.full((_L,), SENT, jnp.int32)
            return carry

        lax.fori_loop(0, (_PEND_MAX - fl) // _L, fbody, 0)
        hb = pl.multiple_of(hptr, 128)
        pltpu.sync_copy(pend.at[pl.ds(0, _PEND_MAX)],
                        plist_hbm.at[w, 0, pl.ds(hb, _PEND_MAX)])
        total = hptr + pend_end
        cntb[...] = total + jnp.zeros((_L,), jnp.int32)
        pltpu.sync_copy(cntb, cnt_hbm.at[w, 0])
        pltpu.sync_copy(hist, deg_hbm.at[w])

    return part_kernel


_SG = 64


def _make_scatter_kernel(EP, NA, D):
    RPW = NA // _NW
    CAPR = EP + _PEND_MAX
    assert NA % _NW == 0 and RPW % 8 == 0 and _GROUP % _SG == 0

    mesh = plsc.VectorSubcoreMesh(core_axis_name="c", subcore_axis_name="s")

    @functools.partial(
        pl.kernel,
        out_type=jax.ShapeDtypeStruct((NA, D), jnp.float32),
        mesh=mesh,
        compiler_params=pltpu.CompilerParams(needs_layout_passes=False),
        scratch_types=[
            pltpu.VMEM((NA // _NW + 8, D), jnp.float32),
            pltpu.VMEM((_SG,), jnp.int32),
            pltpu.VMEM((_SG,), jnp.int32),
            pltpu.VMEM((_SG,), jnp.int32),
            pltpu.VMEM((_SG,), jnp.int32),
            pltpu.VMEM((_SG, D), jnp.float32),
            pltpu.VMEM((_SG, D), jnp.float32),
            pltpu.VMEM((_L,), jnp.int32),
            pltpu.SemaphoreType.DMA,
            pltpu.SemaphoreType.DMA,
        ],
    )
    def scat_kernel(hp_hbm, plist_hbm, cnt_hbm, out_hbm,
                    acc, pgA, pgB, idxA, idxB, gbA, gbB, cbuf, semA, semB):
        c = lax.axis_index("c")
        s = lax.axis_index("s")
        w = s * _NC + c
        lo = w * RPW
        zero16 = jnp.zeros((_L,), jnp.float32)

        def zbody(r, carry):
            for t in range(D // _L):
                acc[r, pl.ds(t * _L, _L)] = zero16
            return carry

        lax.fori_loop(0, RPW + 8, zbody, 0)

        pltpu.sync_copy(cnt_hbm.at[w, 0], cbuf)
        total = cbuf[...][0]
        ngroups = total // _SG

        pg = (pgA, pgB)
        idx = (idxA, idxB)
        gb = (gbA, gbB)
        sem = (semA, semB)

        def fire(g, k):
            o = pl.multiple_of(g * _SG, 64)
            pltpu.sync_copy(plist_hbm.at[w, 0, pl.ds(o, _SG)], pg[k])
            for t in range(_SG // _L):
                pv16 = pg[k][pl.ds(t * _L, _L)]
                idx[k][pl.ds(t * _L, _L)] = pv16 & 16383
            pltpu.async_copy(hp_hbm.at[idx[k]], gb[k], sem[k])

        def wait(k):
            pltpu.make_async_copy(hp_hbm.at[idx[k]], gb[k], sem[k]).wait()

        def accum(k):
            def ebody(u, carry2):
                sb = u * _L
                lvv = pg[k][pl.ds(sb, _L)] >> 14
                for e in range(_L):
                    lv = lvv[e]
                    ge = sb + e
                    for t in range(D // _L):
                        plsc.addupdate(acc.at[lv, pl.ds(t * _L, _L)],
                                       gb[k][ge, pl.ds(t * _L, _L)])
                return carry2

            lax.fori_loop(0, _SG // _L, ebody, 0)

        @pl.when(ngroups > 0)
        def _():
            fire(0, 0)

            def pbody(p, carry):
                g0 = p * 2
                g1 = g0 + 1
                wait(0)

                @pl.when(g1 < ngroups)
                def _():
                    fire(g1, 1)

                accum(0)

                @pl.when(g1 < ngroups)
                def _():
                    wait(1)

                    @pl.when(g1 + 1 < ngroups)
                    def _():
                        fire(g1 + 1, 0)

                    accum(1)

                return carry

            lax.fori_loop(0, (ngroups + 1) // 2, pbody, 0)

        pltpu.sync_copy(acc.at[pl.ds(0, RPW)], out_hbm.at[pl.ds(lo, RPW)])

    return scat_kernel


def _tc1_body(x_ref, w_ref, deg_ref, h1p_ref, dinv_ref):
    dinv = lax.rsqrt(deg_ref[...] + 1.0)
    h = jnp.dot(x_ref[...], w_ref[...], preferred_element_type=jnp.float32)
    h1p_ref[...] = h * dinv
    dinv_ref[...] = dinv


def _tc2_body(agg_ref, h1p_ref, dinv_ref, b0_ref, w1_ref, h2p_ref):
    dinv = dinv_ref[...]
    t = (agg_ref[...] + h1p_ref[...]) * dinv + b0_ref[...]
    z = jnp.maximum(t, 0.0)
    h2p_ref[...] = jnp.dot(z, w1_ref[...],
                           preferred_element_type=jnp.float32) * dinv


def _tc3_body(agg_ref, h2p_ref, dinv_ref, b1_ref, out_ref):
    out_ref[...] = (agg_ref[...] + h2p_ref[...]) * dinv_ref[...] + b1_ref[...]


def kernel(x, edge_index, W0, b0, W1, b1):
    N, Din = x.shape
    E = edge_index.shape[1]
    Dh = W0.shape[1]
    Dout = W1.shape[1]

    NA = ((N + _NW * 8 - 1) // (_NW * 8)) * (_NW * 8)
    EP = _pad_edges(E)
    pad = EP - E
    row = jnp.concatenate([edge_index[0], jnp.zeros((pad,), jnp.int32)])
    col = jnp.concatenate([edge_index[1], jnp.full((pad,), NA, jnp.int32)])

    part_kernel = _make_part_kernel(EP, NA)
    scat1 = _make_scatter_kernel(EP, NA, Dh)
    scat2 = _make_scatter_kernel(EP, NA, Dout)

    plist, cnts, dhists = part_kernel(row, col)
    degs = (jnp.sum(dhists, axis=0)[:NA // _L, :]
            .reshape(NA)[:N].reshape(N, 1))

    BM = 1000
    assert N % BM == 0
    grid = (N // BM,)

    h1p, dinv = pl.pallas_call(
        _tc1_body,
        grid=grid,
        in_specs=[
            pl.BlockSpec((BM, Din), lambda i: (i, 0)),
            pl.BlockSpec((Din, Dh), lambda i: (0, 0)),
            pl.BlockSpec((BM, 1), lambda i: (i, 0)),
        ],
        out_specs=[
            pl.BlockSpec((BM, Dh), lambda i: (i, 0)),
            pl.BlockSpec((BM, 1), lambda i: (i, 0)),
        ],
        out_shape=[
            jax.ShapeDtypeStruct((N, Dh), jnp.float32),
            jax.ShapeDtypeStruct((N, 1), jnp.float32),
        ],
    )(x, W0, degs)

    agg1 = scat1(h1p, plist, cnts)

    h2p = pl.pallas_call(
        _tc2_body,
        grid=grid,
        in_specs=[
            pl.BlockSpec((BM, Dh), lambda i: (i, 0)),
            pl.BlockSpec((BM, Dh), lambda i: (i, 0)),
            pl.BlockSpec((BM, 1), lambda i: (i, 0)),
            pl.BlockSpec((1, Dh), lambda i: (0, 0)),
            pl.BlockSpec((Dh, Dout), lambda i: (0, 0)),
        ],
        out_specs=pl.BlockSpec((BM, Dout), lambda i: (i, 0)),
        out_shape=jax.ShapeDtypeStruct((N, Dout), jnp.float32),
    )(agg1, h1p, dinv, b0.reshape(1, Dh), W1)

    agg2 = scat2(h2p, plist, cnts)

    out = pl.pallas_call(
        _tc3_body,
        grid=grid,
        in_specs=[
            pl.BlockSpec((BM, Dout), lambda i: (i, 0)),
            pl.BlockSpec((BM, Dout), lambda i: (i, 0)),
            pl.BlockSpec((BM, 1), lambda i: (i, 0)),
            pl.BlockSpec((1, Dout), lambda i: (0, 0)),
        ],
        out_specs=pl.BlockSpec((BM, Dout), lambda i: (i, 0)),
        out_shape=jax.ShapeDtypeStruct((N, Dout), jnp.float32),
    )(agg2, h2p, dinv, b1.reshape(1, Dout))

    return out


# speedup vs baseline: 1.2477x; 1.1000x over previous
import functools

import jax
import jax.numpy as jnp
from jax import lax
from jax.experimental import pallas as pl
from jax.experimental.pallas import tpu as pltpu
from jax.experimental.pallas import tpu_sc as plsc

_NC = 2
_NS = 16
_NW = _NC * _NS
_L = 16
_CHUNK = 512
_GROUP = 128
_FLUSH = 1024
_PCAP = 2128
_PEND_MAX = 1536


def _pad_edges(E):
    step = _NW * _CHUNK
    return ((E + step - 1) // step) * step


def _make_part_kernel(EP, NA):
    RPW = NA // _NW
    TRASH = RPW
    SENT = TRASH << 14
    NCH = EP // _CHUNK
    PWCH = NCH // _NW
    HR = NA // _L
    CAPR = EP + _PEND_MAX
    assert EP % (_CHUNK * _NW) == 0 and NA % _NW == 0 and RPW % 8 == 0

    mesh = plsc.VectorSubcoreMesh(core_axis_name="c", subcore_axis_name="s")

    @functools.partial(
        pl.kernel,
        out_type=(
            jax.ShapeDtypeStruct((_NW, 1, CAPR), jnp.int32),
            jax.ShapeDtypeStruct((_NW, 1, _L), jnp.int32),
            jax.ShapeDtypeStruct((_NW, HR + _L, _L), jnp.float32),
        ),
        mesh=mesh,
        compiler_params=pltpu.CompilerParams(needs_layout_passes=False),
        scratch_types=[
            pltpu.VMEM((_CHUNK,), jnp.int32),
            pltpu.VMEM((_CHUNK,), jnp.int32),
            pltpu.VMEM((_PCAP,), jnp.int32),
            pltpu.VMEM((HR + _L, _L), jnp.float32),
            pltpu.VMEM((_L,), jnp.int32),
        ],
    )
    def part_kernel(row_hbm, col_hbm, plist_hbm, cnt_hbm, deg_hbm,
                    rowb, colb, pend, hist, cntb):
        c = lax.axis_index("c")
        s = lax.axis_index("s")
        w = s * _NC + c
        lo = w * RPW
        zero16 = jnp.zeros((_L,), jnp.float32)
        one = jnp.float32(1.0)
        iota = lax.iota(jnp.int32, _L)

        def zbody(r, carry):
            hist[r, :] = zero16
            return carry

        lax.fori_loop(0, HR + _L, zbody, 0)

        def body(i, carry):
            ptr, hptr = carry
            base = pl.multiple_of(i * _CHUNK, 8)
            pltpu.sync_copy(row_hbm.at[pl.ds(base, _CHUNK)], rowb)
            pltpu.sync_copy(col_hbm.at[pl.ds(base, _CHUNK)], colb)

            def sbody(j, p):
                cv = colb[pl.ds(j * _L, _L)]
                rv = rowb[pl.ds(j * _L, _L)]
                lv = cv - lo
                m = (lv >= 0) & (lv < RPW)
                key = jnp.where(m, 0, 1)
                pv = jnp.where(m, rv | (lv << 14), SENT)
                _, sv = plsc.sort_key_val(key, pv)
                pend[pl.ds(p, _L)] = sv
                return p + plsc.all_reduce_population_count(m)[0]

            ptr = lax.fori_loop(0, _CHUNK // _L, sbody, ptr)

            @pl.when((i >= w * PWCH) & (i < (w + 1) * PWCH))
            def _():
                def dbody(j, carry2):
                    cv = colb[pl.ds(j * _L, _L)]
                    for e in range(_L):
                        v = cv[e]
                        onehot = jnp.where(iota == (v & (_L - 1)), one, 0.0)
                        plsc.addupdate(hist.at[v >> 4], onehot)
                    return carry2

                lax.fori_loop(0, _CHUNK // _L, dbody, 0)

            def flush(args):
                p, h = args
                hb = pl.multiple_of(h, 128)
                pltpu.sync_copy(pend.at[pl.ds(0, _FLUSH)],
                                plist_hbm.at[w, 0, pl.ds(hb, _FLUSH)])
                rem = p - _FLUSH
                nrem = (rem + _L - 1) // _L

                def mbody(t, carry3):
                    src = pl.multiple_of(_FLUSH + t * _L, 8)
                    dst = pl.multiple_of(t * _L, 8)
                    pend[pl.ds(dst, _L)] = pend[pl.ds(src, _L)]
                    return carry3

                lax.fori_loop(0, nrem, mbody, 0)
                return rem, h + _FLUSH

            ptr, hptr = lax.cond(ptr >= _FLUSH, flush,
                                 lambda a: a, (ptr, hptr))
            return ptr, hptr

        ptr, hptr = lax.fori_loop(0, NCH, body,
                                  (jnp.int32(0), jnp.int32(0)))

        ngroups = (ptr + _GROUP - 1) // _GROUP
        pend_end = ngroups * _GROUP
        fl = pl.multiple_of((ptr // _L) * _L, 8)
        lane = ptr - fl
        keep = iota < lane
        pend[pl.ds(fl, _L)] = jnp.where(keep, pend[pl.ds(fl, _L)],
                                        jnp.int32(SENT))

        def fbody(t, carry):
            dst = pl.multiple_of(fl + _L + t * _L, 8)
            pend[pl.ds(dst, _L)] = jnp.full((_L,), SENT, jnp.int32)
            return carry

        lax.fori_loop(0, (_PEND_MAX - fl) // _L, fbody, 0)
        hb = pl.multiple_of(hptr, 128)
        pltpu.sync_copy(pend.at[pl.ds(0, _PEND_MAX)],
                        plist_hbm.at[w, 0, pl.ds(hb, _PEND_MAX)])
        total = hptr + pend_end
        cntb[...] = total + jnp.zeros((_L,), jnp.int32)
        pltpu.sync_copy(cntb, cnt_hbm.at[w, 0])
        pltpu.sync_copy(hist, deg_hbm.at[w])

    return part_kernel


_SG = 64


def _make_scatter_kernel(EP, NA, D):
    RPW = NA // _NW
    CAPR = EP + _PEND_MAX
    assert NA % _NW == 0 and RPW % 8 == 0 and _GROUP % _SG == 0

    mesh = plsc.VectorSubcoreMesh(core_axis_name="c", subcore_axis_name="s")

    @functools.partial(
        pl.kernel,
        out_type=jax.ShapeDtypeStruct((NA, D), jnp.float32),
        mesh=mesh,
        compiler_params=pltpu.CompilerParams(needs_layout_passes=False),
        scratch_types=[
            pltpu.VMEM((NA // _NW + 8, D), jnp.float32),
            pltpu.VMEM((_SG,), jnp.int32),
            pltpu.VMEM((_SG,), jnp.int32),
            pltpu.VMEM((_SG,), jnp.int32),
            pltpu.VMEM((_SG,), jnp.int32),
            pltpu.VMEM((_SG, D), jnp.float32),
            pltpu.VMEM((_SG, D), jnp.float32),
            pltpu.VMEM((_L,), jnp.int32),
            pltpu.SemaphoreType.DMA,
            pltpu.SemaphoreType.DMA,
        ],
    )
    def scat_kernel(hp_hbm, plist_hbm, cnt_hbm, out_hbm,
                    acc, pgA, pgB, idxA, idxB, gbA, gbB, cbuf, semA, semB):
        c = lax.axis_index("c")
        s = lax.axis_index("s")
        w = s * _NC + c
        lo = w * RPW
        zero16 = jnp.zeros((_L,), jnp.float32)

        def zbody(r, carry):
            for t in range(D // _L):
                acc[r, pl.ds(t * _L, _L)] = zero16
            return carry

        lax.fori_loop(0, RPW + 8, zbody, 0)

        pltpu.sync_copy(cnt_hbm.at[w, 0], cbuf)
        total = cbuf[...][0]
        ngroups = total // _SG

        pg = (pgA, pgB)
        idx = (idxA, idxB)
        gb = (gbA, gbB)
        sem = (semA, semB)

        def fire(g, k):
            o = pl.multiple_of(g * _SG, 64)
            pltpu.sync_copy(plist_hbm.at[w, 0, pl.ds(o, _SG)], pg[k])
            for t in range(_SG // _L):
                pv16 = pg[k][pl.ds(t * _L, _L)]
                idx[k][pl.ds(t * _L, _L)] = pv16 & 16383
            pltpu.async_copy(hp_hbm.at[idx[k]], gb[k], sem[k])

        def wait(k):
            pltpu.make_async_copy(hp_hbm.at[idx[k]], gb[k], sem[k]).wait()

        def accum(k):
            def ebody(u, carry2):
                sb = u * _L
                lvv = pg[k][pl.ds(sb, _L)] >> 14
                for e in range(_L):
                    lv = lvv[e]
                    ge = sb + e
                    for t in range(D // _L):
                        plsc.addupdate(acc.at[lv, pl.ds(t * _L, _L)],
                                       gb[k][ge, pl.ds(t * _L, _L)])
                return carry2

            lax.fori_loop(0, _SG // _L, ebody, 0)

        @pl.when(ngroups > 0)
        def _():
            fire(0, 0)

            def pbody(p, carry):
                g0 = p * 2
                g1 = g0 + 1
                wait(0)

                @pl.when(g1 < ngroups)
                def _():
                    fire(g1, 1)

                accum(0)

                @pl.when(g1 < ngroups)
                def _():
                    wait(1)

                    @pl.when(g1 + 1 < ngroups)
                    def _():
                        fire(g1 + 1, 0)

                    accum(1)

                return carry

            lax.fori_loop(0, (ngroups + 1) // 2, pbody, 0)

        pltpu.sync_copy(acc.at[pl.ds(0, RPW)], out_hbm.at[pl.ds(lo, RPW)])

    return scat_kernel


def _tc1_body(x_ref, w_ref, deg_ref, h1p_ref, dinv_ref):
    dinv = lax.rsqrt(deg_ref[...] + 1.0)
    h = jnp.dot(x_ref[...], w_ref[...], preferred_element_type=jnp.float32)
    h1p_ref[...] = h * dinv
    dinv_ref[...] = dinv


def _tc2_body(agg_ref, h1p_ref, dinv_ref, b0_ref, w1_ref, h2p_ref):
    dinv = dinv_ref[...]
    t = (agg_ref[...] + h1p_ref[...]) * dinv + b0_ref[...]
    z = jnp.maximum(t, 0.0)
    h2p_ref[...] = jnp.dot(z, w1_ref[...],
                           preferred_element_type=jnp.float32) * dinv


def _tc3_body(agg_ref, h2p_ref, dinv_ref, b1_ref, out_ref):
    out_ref[...] = (agg_ref[...] + h2p_ref[...]) * dinv_ref[...] + b1_ref[...]


def kernel(x, edge_index, W0, b0, W1, b1):
    N, Din = x.shape
    E = edge_index.shape[1]
    Dh = W0.shape[1]
    Dout = W1.shape[1]

    NA = ((N + _NW * 8 - 1) // (_NW * 8)) * (_NW * 8)
    EP = _pad_edges(E)
    pad = EP - E
    row = jnp.concatenate([edge_index[0], jnp.zeros((pad,), jnp.int32)])
    col = jnp.concatenate([edge_index[1], jnp.full((pad,), NA, jnp.int32)])

    part_kernel = _make_part_kernel(EP, NA)
    scat1 = _make_scatter_kernel(EP, NA, Dh)
    scat2 = _make_scatter_kernel(EP, NA, Dout)

    plist, cnts, dhists = part_kernel(row, col)
    degs = (jnp.sum(dhists, axis=0)[:NA // _L, :]
            .reshape(NA)[:N].reshape(N, 1))

    BM = 1000
    assert N % BM == 0
    grid = (N // BM,)

    h1p, dinv = pl.pallas_call(
        _tc1_body,
        grid=grid,
        in_specs=[
            pl.BlockSpec((BM, Din), lambda i: (i, 0)),
            pl.BlockSpec((Din, Dh), lambda i: (0, 0)),
            pl.BlockSpec((BM, 1), lambda i: (i, 0)),
        ],
        out_specs=[
            pl.BlockSpec((BM, Dh), lambda i: (i, 0)),
            pl.BlockSpec((BM, 1), lambda i: (i, 0)),
        ],
        out_shape=[
            jax.ShapeDtypeStruct((N, Dh), jnp.float32),
            jax.ShapeDtypeStruct((N, 1), jnp.float32),
        ],
    )(x, W0, degs)

    agg1 = scat1(h1p, plist, cnts)

    h2p = pl.pallas_call(
        _tc2_body,
        grid=grid,
        in_specs=[
            pl.BlockSpec((BM, Dh), lambda i: (i, 0)),
            pl.BlockSpec((BM, Dh), lambda i: (i, 0)),
            pl.BlockSpec((BM, 1), lambda i: (i, 0)),
            pl.BlockSpec((1, Dh), lambda i: (0, 0)),
            pl.BlockSpec((Dh, Dout), lambda i: (0, 0)),
        ],
        out_specs=pl.BlockSpec((BM, Dout), lambda i: (i, 0)),
        out_shape=jax.ShapeDtypeStruct((N, Dout), jnp.float32),
    )(agg1, h1p, dinv, b0.reshape(1, Dh), W1)

    agg2 = scat2(h2p, plist, cnts)

    out = pl.pallas_call(
        _tc3_body,
        grid=grid,
        in_specs=[
            pl.BlockSpec((BM, Dout), lambda i: (i, 0)),
            pl.BlockSpec((BM, Dout), lambda i: (i, 0)),
            pl.BlockSpec((BM, 1), lambda i: (i, 0)),
            pl.BlockSpec((1, Dout), lambda i: (0, 0)),
        ],
        out_specs=pl.BlockSpec((BM, Dout), lambda i: (i, 0)),
        out_shape=jax.ShapeDtypeStruct((N, Dout), jnp.float32),
    )(agg2, h2p, dinv, b1.reshape(1, Dout))

    return out
